# baseline (speedup 1.0000x reference)
import functools

import jax
import jax.numpy as jnp
from jax import lax
from jax.experimental import pallas as pl
from jax.experimental.pallas import tpu as pltpu
from jax.experimental.pallas import tpu_sc as plsc

BATCH = 16384
EMBED_DIM = 64
NUM_TABLES = 3
NC = 2
NS = 16
NW = NC * NS
B_PER_W = BATCH // NW
CHUNK = 128
N_CHUNKS = B_PER_W // CHUNK

_MESH = plsc.VectorSubcoreMesh(core_axis_name="c", subcore_axis_name="s")


@functools.partial(
    pl.kernel,
    out_type=jax.ShapeDtypeStruct((BATCH, NUM_TABLES * EMBED_DIM), jnp.float32),
    mesh=_MESH,
    scratch_types=[
        pltpu.VMEM((B_PER_W,), jnp.int32),
        pltpu.VMEM((B_PER_W,), jnp.int32),
        pltpu.VMEM((B_PER_W,), jnp.int32),
        pltpu.VMEM((B_PER_W, EMBED_DIM), jnp.float32),
        pltpu.VMEM((B_PER_W, EMBED_DIM), jnp.float32),
        pltpu.VMEM((B_PER_W, EMBED_DIM), jnp.float32),
        pltpu.SemaphoreType.DMA,
    ],
    compiler_params=pltpu.CompilerParams(use_tc_tiling_on_sc=False),
)
def _emb_kernel(iu_hbm, if_hbm, ia_hbm, w_user, w_fans, w_avg, out_hbm,
                idx_u, idx_f, idx_a, rows_u, rows_f, rows_a, sem):
    wid = lax.axis_index("s") * NC + lax.axis_index("c")
    base = wid * B_PER_W
    for src, idx_v in ((iu_hbm, idx_u), (if_hbm, idx_f), (ia_hbm, idx_a)):
        pltpu.sync_copy(src.at[pl.ds(base, B_PER_W)], idx_v)
    copies = []
    for idx_v, table, rows in ((idx_u, w_user, rows_u),
                               (idx_f, w_fans, rows_f),
                               (idx_a, w_avg, rows_a)):
        for c in range(N_CHUNKS):
            copies.append(pltpu.async_copy(
                table.at[idx_v.at[pl.ds(c * CHUNK, CHUNK)]],
                rows.at[pl.ds(c * CHUNK, CHUNK), :],
                sem))
    for cp in copies:
        cp.wait()
    for t, rows in enumerate((rows_u, rows_f, rows_a)):
        pltpu.sync_copy(
            rows,
            out_hbm.at[pl.ds(base, B_PER_W),
                       pl.ds(t * EMBED_DIM, EMBED_DIM)])


def kernel(user_fea, W_user, W_fans, W_avg):
    idx = user_fea.T
    return _emb_kernel(idx[0], idx[1], idx[2], W_user, W_fans, W_avg)
